# baseline (speedup 1.0000x reference)
import functools

import jax
import jax.numpy as jnp
from jax import lax
from jax.experimental import pallas as pl
from jax.experimental.pallas import tpu as pltpu
from jax.experimental.pallas import tpu_sc as plsc

_info = plsc.get_sparse_core_info()
_NC = _info.num_cores
_NS = _info.num_subcores
_NW = _NC * _NS


def _make_sc_batch_fill(B, S, D, dtype, fill_b):
    CH = 32
    NBUF = 2
    rows_per_w = S // _NW
    n_steps = rows_per_w // CH
    mesh = plsc.VectorSubcoreMesh(core_axis_name="c", subcore_axis_name="s")

    @functools.partial(
        pl.kernel,
        mesh=mesh,
        out_type=jax.ShapeDtypeStruct((B, S, D), dtype),
        scratch_types=[pltpu.VMEM((NBUF, CH, D), dtype)]
        + [pltpu.SemaphoreType.DMA] * (2 * NBUF),
    )
    def sc_kernel(table_hbm, out_hbm, buf, *sems):
        rsem = sems[:NBUF]
        wsem = sems[NBUF:]
        w = lax.axis_index("s") * _NC + lax.axis_index("c")
        base = w * rows_per_w

        reads = {}
        writes = {}
        reads[0] = pltpu.async_copy(
            table_hbm.at[pl.ds(base, CH)], buf.at[0], rsem[0]
        )
        for i in range(n_steps):
            reads.pop(i).wait()
            nxt = i + 1
            if nxt < n_steps:
                prev = nxt - NBUF
                if prev in writes:
                    writes.pop(prev).wait()
                reads[nxt] = pltpu.async_copy(
                    table_hbm.at[pl.ds(base + nxt * CH, CH)],
                    buf.at[nxt % NBUF],
                    rsem[nxt % NBUF],
                )
            r0 = base + i * CH
            writes[i] = pltpu.async_copy(
                buf.at[i % NBUF], out_hbm.at[fill_b, pl.ds(r0, CH)], wsem[i % NBUF]
            )
        for h in writes.values():
            h.wait()

    return sc_kernel


def _tc_body(table_ref, _partial_ref, out_ref):
    out_ref[...] = table_ref[...][None]


def kernel(inputs, table):
    B = inputs.shape[0]
    S, D = table.shape
    partial = _make_sc_batch_fill(B, S, D, table.dtype, B - 1)(table)
    BS = 1024
    grid = (S // BS, B - 1)
    out = pl.pallas_call(
        _tc_body,
        grid=grid,
        in_specs=[
            pl.BlockSpec((BS, D), lambda s, b: (s, 0)),
            pl.BlockSpec(memory_space=pl.ANY),
        ],
        out_specs=pl.BlockSpec((1, BS, D), lambda s, b: (b, s, 0)),
        out_shape=jax.ShapeDtypeStruct((B, S, D), table.dtype),
        input_output_aliases={1: 0},
    )(table, partial)
    return out


# speedup vs baseline: 1.2936x; 1.2936x over previous
import functools

import jax
import jax.numpy as jnp
from jax import lax
from jax.experimental import pallas as pl
from jax.experimental.pallas import tpu as pltpu
from jax.experimental.pallas import tpu_sc as plsc

_info = plsc.get_sparse_core_info()
_NC = _info.num_cores
_NS = _info.num_subcores
_NW = _NC * _NS


def _make_sc_batch_fill(B, S, D, dtype, fill_b):
    CH = 32
    NBUF = 2
    rows_per_w = S // _NW
    n_steps = rows_per_w // CH
    mesh = plsc.VectorSubcoreMesh(core_axis_name="c", subcore_axis_name="s")

    @functools.partial(
        pl.kernel,
        mesh=mesh,
        out_type=jax.ShapeDtypeStruct((B, S, D), dtype),
        scratch_types=[pltpu.VMEM((NBUF, CH, D), dtype)]
        + [pltpu.SemaphoreType.DMA] * (2 * NBUF),
    )
    def sc_kernel(table_hbm, out_hbm, buf, *sems):
        rsem = sems[:NBUF]
        wsem = sems[NBUF:]
        w = lax.axis_index("s") * _NC + lax.axis_index("c")
        base = w * rows_per_w

        reads = {}
        writes = {}
        reads[0] = pltpu.async_copy(
            table_hbm.at[pl.ds(base, CH)], buf.at[0], rsem[0]
        )
        for i in range(n_steps):
            reads.pop(i).wait()
            nxt = i + 1
            if nxt < n_steps:
                prev = nxt - NBUF
                if prev in writes:
                    writes.pop(prev).wait()
                reads[nxt] = pltpu.async_copy(
                    table_hbm.at[pl.ds(base + nxt * CH, CH)],
                    buf.at[nxt % NBUF],
                    rsem[nxt % NBUF],
                )
            r0 = base + i * CH
            writes[i] = pltpu.async_copy(
                buf.at[i % NBUF], out_hbm.at[fill_b, pl.ds(r0, CH)], wsem[i % NBUF]
            )
        for h in writes.values():
            h.wait()

    return sc_kernel


def _make_sc_full(B, S, D, dtype):
    CH = 32
    NBUF = 2
    rows_per_w = S // _NW
    n_steps = rows_per_w // CH
    mesh = plsc.VectorSubcoreMesh(core_axis_name="c", subcore_axis_name="s")

    @functools.partial(
        pl.kernel,
        mesh=mesh,
        out_type=jax.ShapeDtypeStruct((B, S, D), dtype),
        scratch_types=[pltpu.VMEM((NBUF, CH, D), dtype)]
        + [pltpu.SemaphoreType.DMA] * (2 * NBUF),
    )
    def sc_kernel(table_hbm, out_hbm, buf, *sems):
        rsem = sems[:NBUF]
        wsem = sems[NBUF:]
        w = lax.axis_index("s") * _NC + lax.axis_index("c")
        base = w * rows_per_w
        reads = {}
        writes = {}
        reads[0] = pltpu.async_copy(
            table_hbm.at[pl.ds(base, CH)], buf.at[0], rsem[0]
        )
        for i in range(n_steps):
            reads.pop(i).wait()
            nxt = i + 1
            if nxt < n_steps:
                prev = nxt - NBUF
                if prev in writes:
                    for h in writes.pop(prev):
                        h.wait()
                reads[nxt] = pltpu.async_copy(
                    table_hbm.at[pl.ds(base + nxt * CH, CH)],
                    buf.at[nxt % NBUF],
                    rsem[nxt % NBUF],
                )
            r0 = base + i * CH
            writes[i] = [
                pltpu.async_copy(
                    buf.at[i % NBUF], out_hbm.at[b, pl.ds(r0, CH)], wsem[i % NBUF]
                )
                for b in range(B)
            ]
        for hs in writes.values():
            for h in hs:
                h.wait()

    return sc_kernel


def _tc_body(table_ref, _partial_ref, out_ref):
    out_ref[...] = table_ref[...][None]


def kernel(inputs, table):
    B = inputs.shape[0]
    S, D = table.shape
    return _make_sc_full(B, S, D, table.dtype)(table)
